# Initial kernel scaffold; baseline (speedup 1.0000x reference)
import jax
import jax.numpy as jnp
from jax.experimental import pallas as pl


def kernel(d1_user_W, d1_item_W, d2_user_W, d2_item_W, d1_adj_idx, d1_adj_val, d2_adj_idx, d2_adj_val, d1_deg, d2_deg):
    raise NotImplementedError("write your pallas kernel here")


# speedup vs baseline: 3.9102x; 3.9102x over previous
import functools

import jax
import jax.numpy as jnp
from jax import lax
from jax.experimental import pallas as pl
from jax.experimental.pallas import tpu as pltpu
from jax.experimental.pallas import tpu_sc as plsc

N_USERS = 100000
D1_N_ITEMS = 40000
D2_N_ITEMS = 30000
EMB = 64
SHARED = 32
N1 = N_USERS + D1_N_ITEMS
N2 = N_USERS + D2_N_ITEMS
E = 1000000

NW = 32
EPW = 31744
EPAD = NW * EPW
T = 7936
NTILES = EPW // T
R = 16384
NCHUNK1 = 9
NCHUNK2 = 8
G = 128
ZR = 32
ROWB = 2000


def _make_spmm(n_chunks):
    npad = R * n_chunks
    mesh = plsc.VectorSubcoreMesh(core_axis_name="c", subcore_axis_name="s")

    nbat = T // G + 1

    @functools.partial(
        pl.kernel,
        out_type=jax.ShapeDtypeStruct((2, npad, EMB), jnp.float32),
        mesh=mesh,
        compiler_params=pltpu.CompilerParams(needs_layout_passes=False,
                                             use_tc_tiling_on_sc=False),
        scratch_types=[
            pltpu.VMEM_SHARED((R, EMB), jnp.float32),
            pltpu.VMEM((T,), jnp.int32),
            pltpu.VMEM((T,), jnp.int32),
            pltpu.VMEM((T,), jnp.float32),
            pltpu.VMEM((nbat, G), jnp.int32),
            pltpu.VMEM((nbat, G), jnp.int32),
            pltpu.VMEM((nbat, G), jnp.float32),
            pltpu.VMEM((G, EMB), jnp.float32),
            pltpu.VMEM((ZR, EMB), jnp.float32),
            pltpu.SemaphoreType.DMA,
        ],
    )
    def spmm(x_hbm, src_hbm, dst_hbm, val_hbm, out_hbm,
             acc, src_t, dst_t, val_t, src_c, dst_c, val_c,
             rows, zblk, sem):
        cax = lax.axis_index("c")
        sid = lax.axis_index("s")
        wid = cax * 16 + sid
        zero16 = jnp.zeros((16,), jnp.float32)
        zero16i = jnp.zeros((16,), jnp.int32)

        def zrow(r, _):
            for g in range(EMB // 16):
                zblk[r, pl.ds(g * 16, 16)] = zero16
            return 0
        lax.fori_loop(0, ZR, zrow, 0)

        rows_per_sub = R // 16

        def chunk_body(c, _):
            lo = c * R

            def zacc(k, _):
                pltpu.sync_copy(zblk, acc.at[pl.ds(sid * rows_per_sub + k * ZR, ZR)])
                return 0
            lax.fori_loop(0, rows_per_sub // ZR, zacc, 0)
            plsc.subcore_barrier()

            def tile_body(t, _):
                base = wid * EPW + t * T
                pltpu.sync_copy(src_hbm.at[pl.ds(base, T)], src_t)
                pltpu.sync_copy(dst_hbm.at[pl.ds(base, T)], dst_t)
                pltpu.sync_copy(val_hbm.at[pl.ds(base, T)], val_t)

                def comp(i, nc):
                    d16 = dst_t[pl.ds(i * 16, 16)]
                    m = (d16 >= lo) & (d16 < lo + R)
                    cs = plsc.cumsum(m.astype(jnp.int32))
                    pos = nc + cs - 1
                    hi = lax.shift_right_logical(pos, 7)
                    lo_ = pos & (G - 1)
                    plsc.store_scatter(src_c, [hi, lo_],
                                       src_t[pl.ds(i * 16, 16)], mask=m)
                    plsc.store_scatter(dst_c, [hi, lo_], d16 - lo, mask=m)
                    plsc.store_scatter(val_c, [hi, lo_],
                                       val_t[pl.ds(i * 16, 16)], mask=m)
                    return nc + cs[15]
                nc = lax.fori_loop(0, T // 16, comp, jnp.int32(0))

                iota16 = lax.iota(jnp.int32, 16)
                for k in range(G // 16):
                    p = nc + k * 16 + iota16
                    hi = lax.shift_right_logical(p, 7)
                    lo_ = p & (G - 1)
                    plsc.store_scatter(src_c, [hi, lo_], zero16i)
                    plsc.store_scatter(dst_c, [hi, lo_], zero16i)
                    plsc.store_scatter(val_c, [hi, lo_], zero16)

                nb = (nc + (G - 1)) // G

                def batch(b, _):
                    pltpu.async_copy(x_hbm.at[src_c.at[b]], rows, sem).wait()

                    def scale16(e16, _):
                        v16 = val_c[b, pl.ds(e16 * 16, 16)]
                        for j in range(16):
                            vj = jnp.full((16,), v16[j], jnp.float32)
                            e = e16 * 16 + j
                            for g in range(EMB // 16):
                                sl = (e, pl.ds(g * 16, 16))
                                rows[sl] = rows[sl] * vj
                        return 0
                    lax.fori_loop(0, G // 16, scale16, 0)

                    pltpu.sync_copy(rows, acc.at[dst_c.at[b]], add=True)
                    return 0
                lax.fori_loop(0, nb, batch, 0)
                return 0
            lax.fori_loop(0, NTILES, tile_body, 0)

            plsc.subcore_barrier()
            r0 = sid * rows_per_sub
            pltpu.sync_copy(acc.at[pl.ds(r0, rows_per_sub)],
                            out_hbm.at[cax, pl.ds(lo + r0, rows_per_sub)])
            plsc.subcore_barrier()
            return 0
        lax.fori_loop(0, n_chunks, chunk_body, 0)

    return spmm


_spmm1 = _make_spmm(NCHUNK1)
_spmm2 = _make_spmm(NCHUNK2)


def _combine_body(xo_r, po_r, xb_r, pb_r, do_r, db_r, y_r, *, ublocks):
    i = pl.program_id(0)
    so = po_r[0] + po_r[1]
    go = xo_r[...] + so + xo_r[...] * so
    sb = pb_r[0] + pb_r[1]
    gb = xb_r[...] + sb + xb_r[...] * sb
    common = do_r[...] * go + db_r[...] * gb
    t = jnp.where(i < ublocks, (common + go) * 0.5, go)
    outs = []
    for h in range(EMB // SHARED):
        v = t[:, h * SHARED:(h + 1) * SHARED]
        n = jnp.maximum(jnp.sqrt(jnp.sum(v * v, axis=-1, keepdims=True)), 1e-12)
        outs.append(v / n)
    y_r[...] = jnp.concatenate(outs, axis=-1)


def _combine(x_own, p_own, x_oth, p_oth, deg_own, deg_oth):
    n = x_own.shape[0]
    grid = n // ROWB
    ublocks = N_USERS // ROWB
    clamp = lambda i: (jnp.minimum(i, ublocks - 1), 0)
    clamp3 = lambda i: (0, jnp.minimum(i, ublocks - 1), 0)
    return pl.pallas_call(
        functools.partial(_combine_body, ublocks=ublocks),
        grid=(grid,),
        in_specs=[
            pl.BlockSpec((ROWB, EMB), lambda i: (i, 0)),
            pl.BlockSpec((2, ROWB, EMB), lambda i: (0, i, 0)),
            pl.BlockSpec((ROWB, EMB), clamp),
            pl.BlockSpec((2, ROWB, EMB), clamp3),
            pl.BlockSpec((ROWB, 1), clamp),
            pl.BlockSpec((ROWB, 1), clamp),
        ],
        out_specs=pl.BlockSpec((ROWB, EMB), lambda i: (i, 0)),
        out_shape=jax.ShapeDtypeStruct((n, EMB), jnp.float32),
    )(x_own, p_own, x_oth, p_oth, deg_own, deg_oth)


def _pad_edges(idx, val):
    pad = EPAD - E
    src = jnp.concatenate([idx[1], jnp.zeros((pad,), jnp.int32)])
    dst = jnp.concatenate([idx[0], jnp.zeros((pad,), jnp.int32)])
    v = jnp.concatenate([val, jnp.zeros((pad,), jnp.float32)])
    return src, dst, v


def kernel(d1_user_W, d1_item_W, d2_user_W, d2_item_W,
           d1_adj_idx, d1_adj_val, d2_adj_idx, d2_adj_val,
           d1_deg, d2_deg):
    x1 = jnp.concatenate([d1_user_W, d1_item_W], 0)
    x2 = jnp.concatenate([d2_user_W, d2_item_W], 0)
    src1, dst1, val1 = _pad_edges(d1_adj_idx, d1_adj_val)
    src2, dst2, val2 = _pad_edges(d2_adj_idx, d2_adj_val)

    l1 = [x1]
    l2 = [x2]
    for _ in range(2):
        p1 = _spmm1(x1, src1, dst1, val1)
        p2 = _spmm2(x2, src2, dst2, val2)
        y1 = _combine(x1, p1, x2, p2, d1_deg, d2_deg)
        y2 = _combine(x2, p2, x1, p1, d2_deg, d1_deg)
        x1, x2 = y1, y2
        l1.append(x1)
        l2.append(x2)

    e1 = jnp.concatenate(l1, -1)
    e2 = jnp.concatenate(l2, -1)
    d1_user = e1[:70000]
    d1_item = e1[N_USERS:]
    d2_user = jnp.concatenate([e2[:20000], e2[70000:N_USERS]], 0)
    d2_item = e2[N_USERS:]
    return d1_user, d1_item, d2_user, d2_item


# speedup vs baseline: 3.9367x; 1.0068x over previous
import functools

import jax
import jax.numpy as jnp
from jax import lax
from jax.experimental import pallas as pl
from jax.experimental.pallas import tpu as pltpu
from jax.experimental.pallas import tpu_sc as plsc

N_USERS = 100000
D1_N_ITEMS = 40000
D2_N_ITEMS = 30000
EMB = 64
SHARED = 32
N1 = N_USERS + D1_N_ITEMS
N2 = N_USERS + D2_N_ITEMS
E = 1000000

NW = 32
EPW = 31744
EPAD = NW * EPW
T = 3968
NTILES = EPW // T
R = 16384
NCHUNK1 = 9
NCHUNK2 = 8
G = 128
GI = 128
ZR = 32
ROWB = 2000


def _make_spmm(n_chunks):
    npad = R * n_chunks
    mesh = plsc.VectorSubcoreMesh(core_axis_name="c", subcore_axis_name="s")

    nbat = T // GI + 2

    @functools.partial(
        pl.kernel,
        out_type=jax.ShapeDtypeStruct((2, npad, EMB), jnp.float32),
        mesh=mesh,
        compiler_params=pltpu.CompilerParams(needs_layout_passes=False,
                                             use_tc_tiling_on_sc=False),
        scratch_types=[
            pltpu.VMEM_SHARED((R, EMB), jnp.float32),
            pltpu.VMEM((T,), jnp.int32),
            pltpu.VMEM((T,), jnp.int32),
            pltpu.VMEM((T,), jnp.float32),
            pltpu.VMEM((nbat, GI), jnp.int32),
            pltpu.VMEM((nbat, GI), jnp.int32),
            pltpu.VMEM((nbat, GI), jnp.float32),
            pltpu.VMEM((G, EMB), jnp.float32),
            pltpu.VMEM((G, EMB), jnp.float32),
            pltpu.VMEM((ZR, EMB), jnp.float32),
            pltpu.SemaphoreType.DMA,
            pltpu.SemaphoreType.DMA,
        ],
    )
    def spmm(x_hbm, src_hbm, dst_hbm, val_hbm, out_hbm,
             acc, src_t, dst_t, val_t, src_c, dst_c, val_c,
             rows_a, rows_b, zblk, sem_a, sem_b):
        cax = lax.axis_index("c")
        sid = lax.axis_index("s")
        wid = cax * 16 + sid
        zero16 = jnp.zeros((16,), jnp.float32)
        zero16i = jnp.zeros((16,), jnp.int32)

        def zrow(r, _):
            for g in range(EMB // 16):
                zblk[r, pl.ds(g * 16, 16)] = zero16
            return 0
        lax.fori_loop(0, ZR, zrow, 0)

        rows_per_sub = R // 16

        def chunk_body(c, _):
            lo = c * R

            def zacc(k, _):
                pltpu.sync_copy(zblk, acc.at[pl.ds(sid * rows_per_sub + k * ZR, ZR)])
                return 0
            lax.fori_loop(0, rows_per_sub // ZR, zacc, 0)
            plsc.subcore_barrier()

            def tile_body(t, _):
                base = wid * EPW + t * T
                pltpu.sync_copy(src_hbm.at[pl.ds(base, T)], src_t)
                pltpu.sync_copy(dst_hbm.at[pl.ds(base, T)], dst_t)
                pltpu.sync_copy(val_hbm.at[pl.ds(base, T)], val_t)

                def comp(i, nc):
                    d16 = dst_t[pl.ds(i * 16, 16)]
                    m = (d16 >= lo) & (d16 < lo + R)
                    cs = plsc.cumsum(m.astype(jnp.int32))
                    pos = nc + cs - 1
                    hi = lax.shift_right_logical(pos, 7)
                    lo_ = pos & (GI - 1)
                    plsc.store_scatter(src_c, [hi, lo_],
                                       src_t[pl.ds(i * 16, 16)], mask=m)
                    plsc.store_scatter(dst_c, [hi, lo_], d16 - lo, mask=m)
                    plsc.store_scatter(val_c, [hi, lo_],
                                       val_t[pl.ds(i * 16, 16)], mask=m)
                    return nc + cs[15]
                nc = lax.fori_loop(0, T // 16, comp, jnp.int32(0))

                iota16 = lax.iota(jnp.int32, 16)
                for k in range(G // 16):
                    p = nc + k * 16 + iota16
                    hi = lax.shift_right_logical(p, 7)
                    lo_ = p & (GI - 1)
                    plsc.store_scatter(src_c, [hi, lo_], zero16i)
                    plsc.store_scatter(dst_c, [hi, lo_], zero16i)
                    plsc.store_scatter(val_c, [hi, lo_], zero16)

                nb = (nc + (G - 1)) // G

                def start(b, rbuf, sem):
                    pltpu.async_copy(x_hbm.at[src_c.at[b]], rbuf, sem)

                def wait(b, rbuf, sem):
                    pltpu.make_async_copy(x_hbm.at[src_c.at[b]], rbuf,
                                          sem).wait()

                def process(b, rbuf):
                    def scale16(e16, _):
                        v16 = val_c[b, pl.ds(e16 * 16, 16)]
                        for j in range(16):
                            vj = jnp.full((16,), v16[j], jnp.float32)
                            e = e16 * 16 + j
                            for g in range(EMB // 16):
                                sl = (e, pl.ds(g * 16, 16))
                                rbuf[sl] = rbuf[sl] * vj
                        return 0
                    lax.fori_loop(0, G // 16, scale16, 0)
                    pltpu.sync_copy(rbuf, acc.at[dst_c.at[b]], add=True)

                @pl.when(nb > 0)
                def _():
                    start(0, rows_a, sem_a)

                def pair(k, _):
                    b0 = 2 * k
                    b1 = b0 + 1

                    @pl.when(b1 < nb)
                    def _():
                        start(b1, rows_b, sem_b)
                    wait(b0, rows_a, sem_a)
                    process(b0, rows_a)

                    @pl.when(b1 < nb)
                    def _():
                        @pl.when(b1 + 1 < nb)
                        def _():
                            start(b1 + 1, rows_a, sem_a)
                        wait(b1, rows_b, sem_b)
                        process(b1, rows_b)
                    return 0
                lax.fori_loop(0, (nb + 1) // 2, pair, 0)
                return 0
            lax.fori_loop(0, NTILES, tile_body, 0)

            plsc.subcore_barrier()
            r0 = sid * rows_per_sub
            pltpu.sync_copy(acc.at[pl.ds(r0, rows_per_sub)],
                            out_hbm.at[cax, pl.ds(lo + r0, rows_per_sub)])
            plsc.subcore_barrier()
            return 0
        lax.fori_loop(0, n_chunks, chunk_body, 0)

    return spmm


_spmm1 = _make_spmm(NCHUNK1)
_spmm2 = _make_spmm(NCHUNK2)


def _combine_body(xo_r, po_r, xb_r, pb_r, do_r, db_r, y_r, *, ublocks):
    i = pl.program_id(0)
    so = po_r[0] + po_r[1]
    go = xo_r[...] + so + xo_r[...] * so
    sb = pb_r[0] + pb_r[1]
    gb = xb_r[...] + sb + xb_r[...] * sb
    common = do_r[...] * go + db_r[...] * gb
    t = jnp.where(i < ublocks, (common + go) * 0.5, go)
    outs = []
    for h in range(EMB // SHARED):
        v = t[:, h * SHARED:(h + 1) * SHARED]
        n = jnp.maximum(jnp.sqrt(jnp.sum(v * v, axis=-1, keepdims=True)), 1e-12)
        outs.append(v / n)
    y_r[...] = jnp.concatenate(outs, axis=-1)


def _combine(x_own, p_own, x_oth, p_oth, deg_own, deg_oth):
    n = x_own.shape[0]
    grid = n // ROWB
    ublocks = N_USERS // ROWB
    clamp = lambda i: (jnp.minimum(i, ublocks - 1), 0)
    clamp3 = lambda i: (0, jnp.minimum(i, ublocks - 1), 0)
    return pl.pallas_call(
        functools.partial(_combine_body, ublocks=ublocks),
        grid=(grid,),
        in_specs=[
            pl.BlockSpec((ROWB, EMB), lambda i: (i, 0)),
            pl.BlockSpec((2, ROWB, EMB), lambda i: (0, i, 0)),
            pl.BlockSpec((ROWB, EMB), clamp),
            pl.BlockSpec((2, ROWB, EMB), clamp3),
            pl.BlockSpec((ROWB, 1), clamp),
            pl.BlockSpec((ROWB, 1), clamp),
        ],
        out_specs=pl.BlockSpec((ROWB, EMB), lambda i: (i, 0)),
        out_shape=jax.ShapeDtypeStruct((n, EMB), jnp.float32),
    )(x_own, p_own, x_oth, p_oth, deg_own, deg_oth)


def _pad_edges(idx, val):
    pad = EPAD - E
    src = jnp.concatenate([idx[1], jnp.zeros((pad,), jnp.int32)])
    dst = jnp.concatenate([idx[0], jnp.zeros((pad,), jnp.int32)])
    v = jnp.concatenate([val, jnp.zeros((pad,), jnp.float32)])
    return src, dst, v


def kernel(d1_user_W, d1_item_W, d2_user_W, d2_item_W,
           d1_adj_idx, d1_adj_val, d2_adj_idx, d2_adj_val,
           d1_deg, d2_deg):
    x1 = jnp.concatenate([d1_user_W, d1_item_W], 0)
    x2 = jnp.concatenate([d2_user_W, d2_item_W], 0)
    src1, dst1, val1 = _pad_edges(d1_adj_idx, d1_adj_val)
    src2, dst2, val2 = _pad_edges(d2_adj_idx, d2_adj_val)

    l1 = [x1]
    l2 = [x2]
    for _ in range(2):
        p1 = _spmm1(x1, src1, dst1, val1)
        p2 = _spmm2(x2, src2, dst2, val2)
        y1 = _combine(x1, p1, x2, p2, d1_deg, d2_deg)
        y2 = _combine(x2, p2, x1, p1, d2_deg, d1_deg)
        x1, x2 = y1, y2
        l1.append(x1)
        l2.append(x2)

    e1 = jnp.concatenate(l1, -1)
    e2 = jnp.concatenate(l2, -1)
    d1_user = e1[:70000]
    d1_item = e1[N_USERS:]
    d2_user = jnp.concatenate([e2[:20000], e2[70000:N_USERS]], 0)
    d2_item = e2[N_USERS:]
    return d1_user, d1_item, d2_user, d2_item


# speedup vs baseline: 4.1157x; 1.0455x over previous
import functools

import jax
import jax.numpy as jnp
from jax import lax
from jax.experimental import pallas as pl
from jax.experimental.pallas import tpu as pltpu
from jax.experimental.pallas import tpu_sc as plsc

N_USERS = 100000
D1_N_ITEMS = 40000
D2_N_ITEMS = 30000
EMB = 64
SHARED = 32
N1 = N_USERS + D1_N_ITEMS
N2 = N_USERS + D2_N_ITEMS
E = 1000000

NW = 32
EPW = 31744
EPAD = NW * EPW
T = 3968
NTILES = EPW // T
R = 16384
NCHUNK1 = 9
NCHUNK2 = 8
G = 128
GI = 128
ZR = 32
ROWB = 2000


def _make_spmm(n_chunks):
    npad = R * n_chunks
    mesh = plsc.VectorSubcoreMesh(core_axis_name="c", subcore_axis_name="s")

    nbat = T // GI + 2

    @functools.partial(
        pl.kernel,
        out_type=jax.ShapeDtypeStruct((2, npad, EMB), jnp.float32),
        mesh=mesh,
        compiler_params=pltpu.CompilerParams(needs_layout_passes=False,
                                             use_tc_tiling_on_sc=False),
        scratch_types=[
            pltpu.VMEM_SHARED((R, EMB), jnp.float32),
            pltpu.VMEM((T,), jnp.int32),
            pltpu.VMEM((T,), jnp.int32),
            pltpu.VMEM((T,), jnp.float32),
            pltpu.VMEM((nbat, GI), jnp.int32),
            pltpu.VMEM((nbat, GI), jnp.int32),
            pltpu.VMEM((nbat, GI), jnp.float32),
            pltpu.VMEM((G, EMB), jnp.float32),
            pltpu.VMEM((G, EMB), jnp.float32),
            pltpu.VMEM((ZR, EMB), jnp.float32),
            pltpu.SemaphoreType.DMA,
            pltpu.SemaphoreType.DMA,
        ],
    )
    def spmm(x_hbm, src_hbm, dst_hbm, val_hbm, out_hbm,
             acc, src_t, dst_t, val_t, src_c, dst_c, val_c,
             rows_a, rows_b, zblk, sem_a, sem_b):
        cax = lax.axis_index("c")
        sid = lax.axis_index("s")
        wid = cax * 16 + sid
        zero16 = jnp.zeros((16,), jnp.float32)
        zero16i = jnp.zeros((16,), jnp.int32)

        def zrow(r, _):
            for g in range(EMB // 16):
                zblk[r, pl.ds(g * 16, 16)] = zero16
            return 0
        lax.fori_loop(0, ZR, zrow, 0)

        rows_per_sub = R // 16

        def chunk_body(c, _):
            lo = c * R

            def zacc(k, _):
                pltpu.sync_copy(zblk, acc.at[pl.ds(sid * rows_per_sub + k * ZR, ZR)])
                return 0
            lax.fori_loop(0, rows_per_sub // ZR, zacc, 0)
            plsc.subcore_barrier()

            def tile_body(t, _):
                base = wid * EPW + t * T
                pltpu.sync_copy(src_hbm.at[pl.ds(base, T)], src_t)
                pltpu.sync_copy(dst_hbm.at[pl.ds(base, T)], dst_t)
                pltpu.sync_copy(val_hbm.at[pl.ds(base, T)], val_t)

                @plsc.parallel_loop(0, T // 16, carry=jnp.int32(0), unroll=4)
                def nc(i, nc):
                    d16 = dst_t[pl.ds(i * 16, 16)]
                    m = (d16 >= lo) & (d16 < lo + R)
                    cs = plsc.cumsum(m.astype(jnp.int32))
                    pos = nc + cs - 1
                    hi = lax.shift_right_logical(pos, 7)
                    lo_ = pos & (GI - 1)
                    plsc.store_scatter(src_c, [hi, lo_],
                                       src_t[pl.ds(i * 16, 16)], mask=m)
                    plsc.store_scatter(dst_c, [hi, lo_], d16 - lo, mask=m)
                    plsc.store_scatter(val_c, [hi, lo_],
                                       val_t[pl.ds(i * 16, 16)], mask=m)
                    return nc + cs[15]

                iota16 = lax.iota(jnp.int32, 16)
                for k in range(G // 16):
                    p = nc + k * 16 + iota16
                    hi = lax.shift_right_logical(p, 7)
                    lo_ = p & (GI - 1)
                    plsc.store_scatter(src_c, [hi, lo_], zero16i)
                    plsc.store_scatter(dst_c, [hi, lo_], zero16i)
                    plsc.store_scatter(val_c, [hi, lo_], zero16)

                nb = (nc + (G - 1)) // G

                def start(b, rbuf, sem):
                    pltpu.async_copy(x_hbm.at[src_c.at[b]], rbuf, sem)

                def wait(b, rbuf, sem):
                    pltpu.make_async_copy(x_hbm.at[src_c.at[b]], rbuf,
                                          sem).wait()

                def process(b, rbuf):
                    @plsc.parallel_loop(0, G // 16, unroll=2)
                    def _(e16):
                        v16 = val_c[b, pl.ds(e16 * 16, 16)]
                        for j in range(16):
                            vj = jnp.full((16,), v16[j], jnp.float32)
                            e = e16 * 16 + j
                            for g in range(EMB // 16):
                                sl = (e, pl.ds(g * 16, 16))
                                rbuf[sl] = rbuf[sl] * vj
                    pltpu.sync_copy(rbuf, acc.at[dst_c.at[b]], add=True)

                @pl.when(nb > 0)
                def _():
                    start(0, rows_a, sem_a)

                def pair(k, _):
                    b0 = 2 * k
                    b1 = b0 + 1

                    @pl.when(b1 < nb)
                    def _():
                        start(b1, rows_b, sem_b)
                    wait(b0, rows_a, sem_a)
                    process(b0, rows_a)

                    @pl.when(b1 < nb)
                    def _():
                        @pl.when(b1 + 1 < nb)
                        def _():
                            start(b1 + 1, rows_a, sem_a)
                        wait(b1, rows_b, sem_b)
                        process(b1, rows_b)
                    return 0
                lax.fori_loop(0, (nb + 1) // 2, pair, 0)
                return 0
            lax.fori_loop(0, NTILES, tile_body, 0)

            plsc.subcore_barrier()
            r0 = sid * rows_per_sub
            pltpu.sync_copy(acc.at[pl.ds(r0, rows_per_sub)],
                            out_hbm.at[cax, pl.ds(lo + r0, rows_per_sub)])
            plsc.subcore_barrier()
            return 0
        lax.fori_loop(0, n_chunks, chunk_body, 0)

    return spmm


_spmm1 = _make_spmm(NCHUNK1)
_spmm2 = _make_spmm(NCHUNK2)


def _combine_body(xo_r, po_r, xb_r, pb_r, do_r, db_r, y_r, *, ublocks):
    i = pl.program_id(0)
    so = po_r[0] + po_r[1]
    go = xo_r[...] + so + xo_r[...] * so
    sb = pb_r[0] + pb_r[1]
    gb = xb_r[...] + sb + xb_r[...] * sb
    common = do_r[...] * go + db_r[...] * gb
    t = jnp.where(i < ublocks, (common + go) * 0.5, go)
    outs = []
    for h in range(EMB // SHARED):
        v = t[:, h * SHARED:(h + 1) * SHARED]
        n = jnp.maximum(jnp.sqrt(jnp.sum(v * v, axis=-1, keepdims=True)), 1e-12)
        outs.append(v / n)
    y_r[...] = jnp.concatenate(outs, axis=-1)


def _combine(x_own, p_own, x_oth, p_oth, deg_own, deg_oth):
    n = x_own.shape[0]
    grid = n // ROWB
    ublocks = N_USERS // ROWB
    clamp = lambda i: (jnp.minimum(i, ublocks - 1), 0)
    clamp3 = lambda i: (0, jnp.minimum(i, ublocks - 1), 0)
    return pl.pallas_call(
        functools.partial(_combine_body, ublocks=ublocks),
        grid=(grid,),
        in_specs=[
            pl.BlockSpec((ROWB, EMB), lambda i: (i, 0)),
            pl.BlockSpec((2, ROWB, EMB), lambda i: (0, i, 0)),
            pl.BlockSpec((ROWB, EMB), clamp),
            pl.BlockSpec((2, ROWB, EMB), clamp3),
            pl.BlockSpec((ROWB, 1), clamp),
            pl.BlockSpec((ROWB, 1), clamp),
        ],
        out_specs=pl.BlockSpec((ROWB, EMB), lambda i: (i, 0)),
        out_shape=jax.ShapeDtypeStruct((n, EMB), jnp.float32),
    )(x_own, p_own, x_oth, p_oth, deg_own, deg_oth)


def _pad_edges(idx, val):
    pad = EPAD - E
    src = jnp.concatenate([idx[1], jnp.zeros((pad,), jnp.int32)])
    dst = jnp.concatenate([idx[0], jnp.zeros((pad,), jnp.int32)])
    v = jnp.concatenate([val, jnp.zeros((pad,), jnp.float32)])
    return src, dst, v


def kernel(d1_user_W, d1_item_W, d2_user_W, d2_item_W,
           d1_adj_idx, d1_adj_val, d2_adj_idx, d2_adj_val,
           d1_deg, d2_deg):
    x1 = jnp.concatenate([d1_user_W, d1_item_W], 0)
    x2 = jnp.concatenate([d2_user_W, d2_item_W], 0)
    src1, dst1, val1 = _pad_edges(d1_adj_idx, d1_adj_val)
    src2, dst2, val2 = _pad_edges(d2_adj_idx, d2_adj_val)

    l1 = [x1]
    l2 = [x2]
    for _ in range(2):
        p1 = _spmm1(x1, src1, dst1, val1)
        p2 = _spmm2(x2, src2, dst2, val2)
        y1 = _combine(x1, p1, x2, p2, d1_deg, d2_deg)
        y2 = _combine(x2, p2, x1, p1, d2_deg, d1_deg)
        x1, x2 = y1, y2
        l1.append(x1)
        l2.append(x2)

    e1 = jnp.concatenate(l1, -1)
    e2 = jnp.concatenate(l2, -1)
    d1_user = e1[:70000]
    d1_item = e1[N_USERS:]
    d2_user = jnp.concatenate([e2[:20000], e2[70000:N_USERS]], 0)
    d2_item = e2[N_USERS:]
    return d1_user, d1_item, d2_user, d2_item


# speedup vs baseline: 4.4018x; 1.0695x over previous
import functools

import jax
import jax.numpy as jnp
from jax import lax
from jax.experimental import pallas as pl
from jax.experimental.pallas import tpu as pltpu
from jax.experimental.pallas import tpu_sc as plsc

N_USERS = 100000
D1_N_ITEMS = 40000
D2_N_ITEMS = 30000
EMB = 64
SHARED = 32
N1 = N_USERS + D1_N_ITEMS
N2 = N_USERS + D2_N_ITEMS
E = 1000000

NW = 32
EPW = 31744
EPAD = NW * EPW
T = 3968
NTILES = EPW // T
R = 20480
NCHUNK1 = 7
NCHUNK2 = 7
G = 128
GI = 128
ZR = 32
ROWB = 2000


def _make_spmm(n_chunks):
    npad = R * n_chunks
    mesh = plsc.VectorSubcoreMesh(core_axis_name="c", subcore_axis_name="s")

    nbat = T // GI + 2

    @functools.partial(
        pl.kernel,
        out_type=jax.ShapeDtypeStruct((2, npad, EMB), jnp.float32),
        mesh=mesh,
        compiler_params=pltpu.CompilerParams(needs_layout_passes=False,
                                             use_tc_tiling_on_sc=False),
        scratch_types=[
            pltpu.VMEM_SHARED((R, EMB), jnp.float32),
            pltpu.VMEM((T,), jnp.int32),
            pltpu.VMEM((T,), jnp.int32),
            pltpu.VMEM((T,), jnp.float32),
            pltpu.VMEM((nbat, GI), jnp.int32),
            pltpu.VMEM((nbat, GI), jnp.int32),
            pltpu.VMEM((nbat, GI), jnp.float32),
            pltpu.VMEM((G, EMB), jnp.float32),
            pltpu.VMEM((G, EMB), jnp.float32),
            pltpu.VMEM((ZR, EMB), jnp.float32),
            pltpu.SemaphoreType.DMA,
            pltpu.SemaphoreType.DMA,
        ],
    )
    def spmm(x_hbm, src_hbm, dst_hbm, val_hbm, out_hbm,
             acc, src_t, dst_t, val_t, src_c, dst_c, val_c,
             rows_a, rows_b, zblk, sem_a, sem_b):
        cax = lax.axis_index("c")
        sid = lax.axis_index("s")
        wid = cax * 16 + sid
        zero16 = jnp.zeros((16,), jnp.float32)
        zero16i = jnp.zeros((16,), jnp.int32)

        def zrow(r, _):
            for g in range(EMB // 16):
                zblk[r, pl.ds(g * 16, 16)] = zero16
            return 0
        lax.fori_loop(0, ZR, zrow, 0)

        rows_per_sub = R // 16

        def chunk_body(c, _):
            lo = c * R

            def zacc(k, _):
                pltpu.sync_copy(zblk, acc.at[pl.ds(sid * rows_per_sub + k * ZR, ZR)])
                return 0
            lax.fori_loop(0, rows_per_sub // ZR, zacc, 0)
            plsc.subcore_barrier()

            def tile_body(t, _):
                base = wid * EPW + t * T
                pltpu.sync_copy(src_hbm.at[pl.ds(base, T)], src_t)
                pltpu.sync_copy(dst_hbm.at[pl.ds(base, T)], dst_t)
                pltpu.sync_copy(val_hbm.at[pl.ds(base, T)], val_t)

                @plsc.parallel_loop(0, T // 16, carry=jnp.int32(0), unroll=8)
                def nc(i, nc):
                    d16 = dst_t[pl.ds(i * 16, 16)]
                    m = (d16 >= lo) & (d16 < lo + R)
                    cs = plsc.cumsum(m.astype(jnp.int32))
                    pos = nc + cs - 1
                    hi = lax.shift_right_logical(pos, 7)
                    lo_ = pos & (GI - 1)
                    plsc.store_scatter(src_c, [hi, lo_],
                                       src_t[pl.ds(i * 16, 16)], mask=m)
                    plsc.store_scatter(dst_c, [hi, lo_], d16 - lo, mask=m)
                    plsc.store_scatter(val_c, [hi, lo_],
                                       val_t[pl.ds(i * 16, 16)], mask=m)
                    return nc + cs[15]

                iota16 = lax.iota(jnp.int32, 16)
                for k in range(G // 16):
                    p = nc + k * 16 + iota16
                    hi = lax.shift_right_logical(p, 7)
                    lo_ = p & (GI - 1)
                    plsc.store_scatter(src_c, [hi, lo_], zero16i)
                    plsc.store_scatter(dst_c, [hi, lo_], zero16i)
                    plsc.store_scatter(val_c, [hi, lo_], zero16)

                nb = (nc + (G - 1)) // G

                def start(b, rbuf, sem):
                    pltpu.async_copy(x_hbm.at[src_c.at[b]], rbuf, sem)

                def wait(b, rbuf, sem):
                    pltpu.make_async_copy(x_hbm.at[src_c.at[b]], rbuf,
                                          sem).wait()

                def process(b, rbuf):
                    @plsc.parallel_loop(0, G // 16, unroll=4)
                    def _(e16):
                        v16 = val_c[b, pl.ds(e16 * 16, 16)]
                        for j in range(16):
                            vj = jnp.full((16,), v16[j], jnp.float32)
                            e = e16 * 16 + j
                            for g in range(EMB // 16):
                                sl = (e, pl.ds(g * 16, 16))
                                rbuf[sl] = rbuf[sl] * vj
                    pltpu.sync_copy(rbuf, acc.at[dst_c.at[b]], add=True)

                @pl.when(nb > 0)
                def _():
                    start(0, rows_a, sem_a)

                def pair(k, _):
                    b0 = 2 * k
                    b1 = b0 + 1

                    @pl.when(b1 < nb)
                    def _():
                        start(b1, rows_b, sem_b)
                    wait(b0, rows_a, sem_a)
                    process(b0, rows_a)

                    @pl.when(b1 < nb)
                    def _():
                        @pl.when(b1 + 1 < nb)
                        def _():
                            start(b1 + 1, rows_a, sem_a)
                        wait(b1, rows_b, sem_b)
                        process(b1, rows_b)
                    return 0
                lax.fori_loop(0, (nb + 1) // 2, pair, 0)
                return 0
            lax.fori_loop(0, NTILES, tile_body, 0)

            plsc.subcore_barrier()
            r0 = sid * rows_per_sub
            pltpu.sync_copy(acc.at[pl.ds(r0, rows_per_sub)],
                            out_hbm.at[cax, pl.ds(lo + r0, rows_per_sub)])
            plsc.subcore_barrier()
            return 0
        lax.fori_loop(0, n_chunks, chunk_body, 0)

    return spmm


_spmm1 = _make_spmm(NCHUNK1)
_spmm2 = _make_spmm(NCHUNK2)


def _combine_body(xo_r, po_r, xb_r, pb_r, do_r, db_r, y_r, *, ublocks):
    i = pl.program_id(0)
    so = po_r[0] + po_r[1]
    go = xo_r[...] + so + xo_r[...] * so
    sb = pb_r[0] + pb_r[1]
    gb = xb_r[...] + sb + xb_r[...] * sb
    common = do_r[...] * go + db_r[...] * gb
    t = jnp.where(i < ublocks, (common + go) * 0.5, go)
    outs = []
    for h in range(EMB // SHARED):
        v = t[:, h * SHARED:(h + 1) * SHARED]
        n = jnp.maximum(jnp.sqrt(jnp.sum(v * v, axis=-1, keepdims=True)), 1e-12)
        outs.append(v / n)
    y_r[...] = jnp.concatenate(outs, axis=-1)


def _combine(x_own, p_own, x_oth, p_oth, deg_own, deg_oth):
    n = x_own.shape[0]
    grid = n // ROWB
    ublocks = N_USERS // ROWB
    clamp = lambda i: (jnp.minimum(i, ublocks - 1), 0)
    clamp3 = lambda i: (0, jnp.minimum(i, ublocks - 1), 0)
    return pl.pallas_call(
        functools.partial(_combine_body, ublocks=ublocks),
        grid=(grid,),
        in_specs=[
            pl.BlockSpec((ROWB, EMB), lambda i: (i, 0)),
            pl.BlockSpec((2, ROWB, EMB), lambda i: (0, i, 0)),
            pl.BlockSpec((ROWB, EMB), clamp),
            pl.BlockSpec((2, ROWB, EMB), clamp3),
            pl.BlockSpec((ROWB, 1), clamp),
            pl.BlockSpec((ROWB, 1), clamp),
        ],
        out_specs=pl.BlockSpec((ROWB, EMB), lambda i: (i, 0)),
        out_shape=jax.ShapeDtypeStruct((n, EMB), jnp.float32),
    )(x_own, p_own, x_oth, p_oth, deg_own, deg_oth)


def _pad_edges(idx, val):
    pad = EPAD - E
    src = jnp.concatenate([idx[1], jnp.zeros((pad,), jnp.int32)])
    dst = jnp.concatenate([idx[0], jnp.zeros((pad,), jnp.int32)])
    v = jnp.concatenate([val, jnp.zeros((pad,), jnp.float32)])
    return src, dst, v


def kernel(d1_user_W, d1_item_W, d2_user_W, d2_item_W,
           d1_adj_idx, d1_adj_val, d2_adj_idx, d2_adj_val,
           d1_deg, d2_deg):
    x1 = jnp.concatenate([d1_user_W, d1_item_W], 0)
    x2 = jnp.concatenate([d2_user_W, d2_item_W], 0)
    src1, dst1, val1 = _pad_edges(d1_adj_idx, d1_adj_val)
    src2, dst2, val2 = _pad_edges(d2_adj_idx, d2_adj_val)

    l1 = [x1]
    l2 = [x2]
    for _ in range(2):
        p1 = _spmm1(x1, src1, dst1, val1)
        p2 = _spmm2(x2, src2, dst2, val2)
        y1 = _combine(x1, p1, x2, p2, d1_deg, d2_deg)
        y2 = _combine(x2, p2, x1, p1, d2_deg, d1_deg)
        x1, x2 = y1, y2
        l1.append(x1)
        l2.append(x2)

    e1 = jnp.concatenate(l1, -1)
    e2 = jnp.concatenate(l2, -1)
    d1_user = e1[:70000]
    d1_item = e1[N_USERS:]
    d2_user = jnp.concatenate([e2[:20000], e2[70000:N_USERS]], 0)
    d2_item = e2[N_USERS:]
    return d1_user, d1_item, d2_user, d2_item
